# baseline (speedup 1.0000x reference)
import functools

import jax
import jax.numpy as jnp
from jax import lax
from jax.experimental import pallas as pl
from jax.experimental.pallas import tpu as pltpu
from jax.experimental.pallas import tpu_sc as plsc

_HB = 16


def _seed_body(t_ref, emb_ref, out_ref):
    del t_ref
    out_ref[...] = jax.lax.broadcast_in_dim(emb_ref[0], out_ref.shape, (1, 2))


def _make_sc_kernel(B, C, H, W):
    info = plsc.get_sparse_core_info()
    NC, NS = info.num_cores, info.num_subcores
    NW = NC * NS
    CW = C // NW
    NK = H // _HB
    mesh = plsc.VectorSubcoreMesh(core_axis_name="c", subcore_axis_name="s")

    @functools.partial(
        pl.kernel,
        mesh=mesh,
        out_type=jax.ShapeDtypeStruct((B, C, H, W), jnp.float32),
        scratch_types=[
            pltpu.VMEM((1, CW, _HB, W), jnp.float32),
            pltpu.SemaphoreType.DMA,
        ],
    )
    def k(seed_hbm, out_hbm, tile_v, sem):
        wid = lax.axis_index("s") * NC + lax.axis_index("c")
        c0 = wid * CW
        pltpu.sync_copy(seed_hbm.at[pl.ds(0, 1), pl.ds(c0, CW)], tile_v)
        copies = [
            pltpu.make_async_copy(
                tile_v,
                out_hbm.at[
                    pl.ds(b, 1), pl.ds(c0, CW), pl.ds(kk * _HB, _HB), pl.ds(0, W)
                ],
                sem,
            )
            for b in range(B)
            for kk in range(NK)
        ]
        for cp in copies:
            cp.start()
        for cp in copies:
            cp.wait()

    return k


def kernel(x, t, embeddings):
    B, _, H, W = x.shape
    C = embeddings.shape[1]
    t_arr = jnp.asarray(t, jnp.int32).reshape((1,))
    emb3 = embeddings.reshape(embeddings.shape[0], C, 1)
    grid_spec = pltpu.PrefetchScalarGridSpec(
        num_scalar_prefetch=1,
        grid=(1,),
        in_specs=[pl.BlockSpec((1, C, 1), lambda i, tr: (tr[0], 0, 0))],
        out_specs=pl.BlockSpec((1, C, _HB, W), lambda i, tr: (0, 0, 0, 0)),
    )
    seed = pl.pallas_call(
        _seed_body,
        grid_spec=grid_spec,
        out_shape=jax.ShapeDtypeStruct((1, C, _HB, W), jnp.float32),
    )(t_arr, emb3)
    sc_k = _make_sc_kernel(B, C, H, W)
    return sc_k(seed)


# speedup vs baseline: 1.0055x; 1.0055x over previous
import functools

import jax
import jax.numpy as jnp
from jax import lax
from jax.experimental import pallas as pl
from jax.experimental.pallas import tpu as pltpu
from jax.experimental.pallas import tpu_sc as plsc

_HB = 16


def _seed_body(t_ref, emb_ref, out_ref):
    del t_ref
    out_ref[...] = jax.lax.broadcast_in_dim(emb_ref[0], out_ref.shape, (1, 2))


def _make_sc_kernel(B, C, H, W):
    info = plsc.get_sparse_core_info()
    NC, NS = info.num_cores, info.num_subcores
    NW = NC * NS
    CW = C // NW
    NK = H // _HB
    mesh = plsc.VectorSubcoreMesh(core_axis_name="c", subcore_axis_name="s")

    @functools.partial(
        pl.kernel,
        mesh=mesh,
        out_type=jax.ShapeDtypeStruct((B, C, H, W), jnp.float32),
        compiler_params=pltpu.CompilerParams(use_tc_tiling_on_sc=True),
        scratch_types=[
            pltpu.VMEM((1, CW, _HB, W), jnp.float32),
            pltpu.SemaphoreType.DMA,
        ],
    )
    def k(seed_hbm, out_hbm, tile_v, sem):
        wid = lax.axis_index("s") * NC + lax.axis_index("c")
        c0 = wid * CW
        pltpu.sync_copy(seed_hbm.at[pl.ds(0, 1), pl.ds(c0, CW)], tile_v)
        copies = [
            pltpu.make_async_copy(
                tile_v,
                out_hbm.at[
                    pl.ds(b, 1), pl.ds(c0, CW), pl.ds(kk * _HB, _HB), pl.ds(0, W)
                ],
                sem,
            )
            for b in range(B)
            for kk in range(NK)
        ]
        for cp in copies:
            cp.start()
        for cp in copies:
            cp.wait()

    return k


def kernel(x, t, embeddings):
    B, _, H, W = x.shape
    C = embeddings.shape[1]
    t_arr = jnp.asarray(t, jnp.int32).reshape((1,))
    emb3 = embeddings.reshape(embeddings.shape[0], C, 1)
    grid_spec = pltpu.PrefetchScalarGridSpec(
        num_scalar_prefetch=1,
        grid=(1,),
        in_specs=[pl.BlockSpec((1, C, 1), lambda i, tr: (tr[0], 0, 0))],
        out_specs=pl.BlockSpec((1, C, _HB, W), lambda i, tr: (0, 0, 0, 0)),
    )
    seed = pl.pallas_call(
        _seed_body,
        grid_spec=grid_spec,
        out_shape=jax.ShapeDtypeStruct((1, C, _HB, W), jnp.float32),
    )(t_arr, emb3)
    sc_k = _make_sc_kernel(B, C, H, W)
    return sc_k(seed)


# speedup vs baseline: 5.1406x; 5.1125x over previous
import jax
import jax.numpy as jnp
from jax.experimental import pallas as pl
from jax.experimental.pallas import tpu as pltpu

_HB = 16


def _body(t_ref, emb_ref, out_ref, row_ref, sem_ref):
    cp = pltpu.make_async_copy(
        emb_ref.at[pl.ds(t_ref[0], 1)], row_ref, sem_ref
    )
    cp.start()
    cp.wait()
    out_ref[...] = jax.lax.broadcast_in_dim(row_ref[0], out_ref.shape, (3,))


def kernel(x, t, embeddings):
    B, _, H, W = x.shape
    C = embeddings.shape[1]
    t_arr = jnp.asarray(t, jnp.int32).reshape((1,))
    grid_spec = pltpu.PrefetchScalarGridSpec(
        num_scalar_prefetch=1,
        grid=(B, H // _HB),
        in_specs=[pl.BlockSpec(memory_space=pl.ANY)],
        out_specs=pl.BlockSpec((1, _HB, W, C), lambda b, i, tr: (b, i, 0, 0)),
        scratch_shapes=[
            pltpu.VMEM((1, C), jnp.float32),
            pltpu.SemaphoreType.DMA,
        ],
    )
    out = pl.pallas_call(
        _body,
        grid_spec=grid_spec,
        out_shape=jax.ShapeDtypeStruct((B, H, W, C), jnp.float32),
        compiler_params=pltpu.CompilerParams(
            dimension_semantics=("parallel", "parallel"),
        ),
    )(t_arr, embeddings)
    return out.transpose(0, 3, 1, 2)


# speedup vs baseline: 6.9016x; 1.3426x over previous
import jax
import jax.numpy as jnp
from jax.experimental import pallas as pl
from jax.experimental.pallas import tpu as pltpu

_HB = 16


def _body(t_ref, emb_ref, out_ref):
    row = emb_ref[0, pl.ds(t_ref[0] % 8, 1), :]
    out_ref[...] = jax.lax.broadcast_in_dim(row, out_ref.shape, (0, 3))


def kernel(x, t, embeddings):
    B, _, H, W = x.shape
    T, C = embeddings.shape
    t_arr = jnp.asarray(t, jnp.int32).reshape((1,))
    emb3 = embeddings.reshape(T // 8, 8, C)
    grid_spec = pltpu.PrefetchScalarGridSpec(
        num_scalar_prefetch=1,
        grid=(B, H // _HB),
        in_specs=[pl.BlockSpec((1, 8, C), lambda b, i, tr: (tr[0] // 8, 0, 0))],
        out_specs=pl.BlockSpec((1, _HB, W, C), lambda b, i, tr: (b, i, 0, 0)),
    )
    out = pl.pallas_call(
        _body,
        grid_spec=grid_spec,
        out_shape=jax.ShapeDtypeStruct((B, H, W, C), jnp.float32),
        compiler_params=pltpu.CompilerParams(
            dimension_semantics=("parallel", "parallel"),
        ),
    )(t_arr, emb3)
    return out.transpose(0, 3, 1, 2)
